# Initial kernel scaffold; baseline (speedup 1.0000x reference)
import jax
import jax.numpy as jnp
from jax.experimental import pallas as pl


def kernel(pred_labels_mask, gt_labels_mask):
    raise NotImplementedError("write your pallas kernel here")


# speedup vs baseline: 32.1465x; 32.1465x over previous
import functools

import jax
import jax.numpy as jnp
import numpy as np
from jax import lax
from jax.experimental import pallas as pl
from jax.experimental.pallas import tpu as pltpu
from jax.experimental.pallas import tpu_sc as plsc

jax.config.update("jax_enable_x64", True)

G = 32
P = 32
N = 512 * 512
NC = 2
NS = 16
L = 16
NW = NC * NS
CHUNK = N // NW
VECS = CHUNK // L
BINS = G * P


@functools.partial(
    pl.kernel,
    mesh=plsc.VectorSubcoreMesh(core_axis_name="c", subcore_axis_name="s"),
    out_type=jax.ShapeDtypeStruct((NC, 8, 128), jnp.int32),
    compiler_params=pltpu.CompilerParams(needs_layout_passes=False),
    scratch_types=[
        pltpu.VMEM((CHUNK,), jnp.int32),
        pltpu.VMEM((CHUNK,), jnp.int32),
        pltpu.VMEM((L * BINS,), jnp.int32),
        pltpu.VMEM((8, 128), jnp.int32),
        pltpu.VMEM((NS, 8, 128), jnp.int32),
        pltpu.VMEM((8, 128), jnp.int32),
        pltpu.VMEM_SHARED((NS, 8, 128), jnp.int32),
    ],
)
def _confusion(gt_hbm, pred_hbm, out_hbm, gt_v, pred_v, hist_v, merged_v,
               all_v, final_v, shared):
    c = lax.axis_index("c")
    s = lax.axis_index("s")
    wid = s * np.int32(NC) + c
    base = pl.multiple_of(wid * np.int32(CHUNK), CHUNK)

    pltpu.sync_copy(gt_hbm.at[pl.ds(base, CHUNK)], gt_v)
    pltpu.sync_copy(pred_hbm.at[pl.ds(base, CHUNK)], pred_v)

    zeros = jnp.zeros((L,), jnp.int32)

    def zero_body(i, off):
        o = pl.multiple_of(off, L)
        hist_v[pl.ds(o, L)] = zeros
        return off + np.int32(L)

    lax.fori_loop(0, (L * BINS) // L, zero_body, jnp.int32(0))

    lane_off = lax.iota(jnp.int32, L) * np.int32(BINS)
    ones = jnp.ones((L,), jnp.int32)

    def count_body(i, off):
        o = pl.multiple_of(off, L)
        g = gt_v[pl.ds(o, L)]
        p = pred_v[pl.ds(o, L)]
        key = g * np.int32(P) + p + lane_off
        plsc.addupdate_scatter(hist_v, [key], ones)
        return off + np.int32(L)

    lax.fori_loop(0, VECS, count_body, jnp.int32(0))

    for j in range(BINS // L):
        acc = hist_v[pl.ds(j * L, L)]
        for l in range(1, L):
            acc = acc + hist_v[pl.ds(l * BINS + j * L, L)]
        merged_v[j // 8, pl.ds((j % 8) * L, L)] = acc

    pltpu.sync_copy(merged_v, shared.at[s])
    plsc.subcore_barrier()

    @pl.when(s == np.int32(0))
    def _finalize():
        pltpu.sync_copy(shared, all_v)
        for j in range(BINS // L):
            r, col = j // 8, (j % 8) * L
            acc = all_v[0, r, pl.ds(col, L)]
            for t in range(1, NS):
                acc = acc + all_v[t, r, pl.ds(col, L)]
            final_v[r, pl.ds(col, L)] = acc
        pltpu.sync_copy(final_v, out_hbm.at[c])


def kernel(pred_labels_mask, gt_labels_mask):
    pred = pred_labels_mask.reshape(-1).astype(jnp.int32)
    gt = gt_labels_mask.reshape(-1).astype(jnp.int32)
    parts = _confusion(gt, pred)
    inter = (parts[0] + parts[1]).reshape(G, P)

    pred_sizes = inter.sum(axis=0)
    gt_sizes = inter.sum(axis=1)
    labels = jnp.arange(P)
    cand = (labels >= 1) & (pred_sizes > 0)
    inter_f = inter.astype(jnp.float64)
    pred_f = pred_sizes.astype(jnp.float64)
    gt_f = gt_sizes.astype(jnp.float64)
    seg_vals = []
    for g in range(1, G):
        gt_size = gt_f[g]
        r_and_s = inter_f[g]
        denom = gt_size + pred_f - r_and_s
        safe_denom = jnp.where(denom > 0, denom, 1.0)
        j = jnp.where(denom > 0, r_and_s / safe_denom, 0.0)
        valid = cand & (r_and_s > 0.5 * gt_size)
        j_masked = jnp.where(valid, j, 0.0)
        max_j = jnp.max(j_masked)
        match = jnp.argmax(j_masked)
        has_match = max_j > 0.0
        cand = cand & ~(has_match & (labels == match))
        seg_vals.append(max_j.astype(jnp.float32))
    seg_arr = jnp.stack(seg_vals)
    seg_avg = seg_arr.mean()
    return (seg_avg, seg_arr)


# speedup vs baseline: 314.0120x; 9.7682x over previous
import functools

import jax
import jax.numpy as jnp
import numpy as np
from jax import lax
from jax.experimental import pallas as pl
from jax.experimental.pallas import tpu as pltpu
from jax.experimental.pallas import tpu_sc as plsc

jax.config.update("jax_enable_x64", True)

G = 32
P = 32
N = 512 * 512
NC = 2
NS = 16
L = 16
NW = NC * NS
CHUNK = N // NW
VECS = CHUNK // L
BINS = G * P


@functools.partial(
    pl.kernel,
    mesh=plsc.VectorSubcoreMesh(core_axis_name="c", subcore_axis_name="s"),
    out_type=jax.ShapeDtypeStruct((NC, 1, BINS), jnp.int32),
    compiler_params=pltpu.CompilerParams(needs_layout_passes=False),
    scratch_types=[
        pltpu.VMEM((CHUNK,), jnp.int32),
        pltpu.VMEM((CHUNK,), jnp.int32),
        pltpu.VMEM((L * BINS,), jnp.int32),
        pltpu.VMEM((1, BINS), jnp.int32),
        pltpu.VMEM((NS, 1, 128), jnp.int32),
        pltpu.VMEM((128,), jnp.int32),
        pltpu.VMEM_SHARED((NS, 1, BINS), jnp.int32),
    ],
)
def _confusion(gt_hbm, pred_hbm, out_hbm, gt_v, pred_v, hist_v, merged_v,
               all_v, final_v, shared):
    c = lax.axis_index("c")
    s = lax.axis_index("s")
    wid = s * np.int32(NC) + c
    base = pl.multiple_of(wid * np.int32(CHUNK), CHUNK)

    pltpu.sync_copy(gt_hbm.at[pl.ds(base, CHUNK)], gt_v)
    pltpu.sync_copy(pred_hbm.at[pl.ds(base, CHUNK)], pred_v)

    zeros = jnp.zeros((L,), jnp.int32)

    def zero_body(i, off):
        o = pl.multiple_of(off, 8 * L)
        for u in range(8):
            hist_v[pl.ds(o + u * L, L)] = zeros
        return off + np.int32(8 * L)

    lax.fori_loop(0, (L * BINS) // (8 * L), zero_body, jnp.int32(0))

    lane_off = lax.iota(jnp.int32, L) * np.int32(BINS)
    ones = jnp.ones((L,), jnp.int32)

    def count_body(i, off):
        o = pl.multiple_of(off, 8 * L)
        for u in range(8):
            g = gt_v[pl.ds(o + u * L, L)]
            p = pred_v[pl.ds(o + u * L, L)]
            key = g * np.int32(P) + p + lane_off
            plsc.addupdate_scatter(hist_v, [key], ones)
        return off + np.int32(8 * L)

    lax.fori_loop(0, VECS // 8, count_body, jnp.int32(0))

    def merge_body(i, off):
        o = pl.multiple_of(off, L)
        acc = hist_v[pl.ds(o, L)]
        for l in range(1, L):
            acc = acc + hist_v[pl.ds(o + l * BINS, L)]
        merged_v[0, pl.ds(o, L)] = acc
        return off + np.int32(L)

    lax.fori_loop(0, BINS // L, merge_body, jnp.int32(0))

    pltpu.sync_copy(merged_v, shared.at[s])
    plsc.subcore_barrier()

    @pl.when(s < np.int32(BINS // 128))
    def _finalize():
        col = pl.multiple_of(s * np.int32(128), 128)
        pltpu.sync_copy(shared.at[:, :, pl.ds(col, 128)], all_v)
        for q in range(128 // L):
            acc = all_v[0, 0, pl.ds(q * L, L)]
            for t in range(1, NS):
                acc = acc + all_v[t, 0, pl.ds(q * L, L)]
            final_v[pl.ds(q * L, L)] = acc
        pltpu.sync_copy(final_v, out_hbm.at[c, np.int32(0), pl.ds(col, 128)])


def _exact_div_limb(x, b, bsafe_f):
    q = (x.astype(jnp.float32) / bsafe_f).astype(jnp.int32)
    t = q * b
    q = jnp.where(t > x, q - np.int32(1), q)
    q = jnp.where(q * b + b <= x, q + np.int32(1), q)
    return q, x - q * b


def _greedy_body(parts_ref, out_ref, avg_ref):
    inter = parts_ref[0] + parts_ref[1]
    pred_sizes = jnp.sum(inter, axis=0, keepdims=True, dtype=jnp.int32)
    gt_sizes = jnp.sum(inter, axis=1, keepdims=True, dtype=jnp.int32)
    denom = gt_sizes + pred_sizes - inter
    vstat = inter + inter > gt_sizes
    bsafe_f = jnp.where(denom > np.int32(0), denom, np.int32(1)).astype(jnp.float32)
    l1, r1 = _exact_div_limb(inter << np.int32(12), denom, bsafe_f)
    l2, r2 = _exact_div_limb(r1 << np.int32(12), denom, bsafe_f)
    l3, _ = _exact_div_limb(r2 << np.int32(12), denom, bsafe_f)
    lane = lax.broadcasted_iota(jnp.int32, (1, P), 1)
    hi_m = jnp.where(vstat, (l1 << np.int32(12)) + l2, np.int32(-1))
    lo_m = jnp.where(vstat, (l3 << np.int32(5)) + (np.int32(P - 1) - lane),
                     np.int32(-1))

    cand = (lane >= np.int32(1)) & (pred_sizes > np.int32(0))
    seg_hi = jnp.full((1, P), np.int32(-1))
    seg_lo = jnp.full((1, P), np.int32(-1))
    for g in range(1, G):
        onehot = lane == np.int32(g - 1)
        hi = jnp.where(cand, hi_m[g:g + 1, :], np.int32(-1))
        best_hi = jnp.max(hi, axis=1, keepdims=True)
        m2 = cand & (hi == best_hi)
        lo = lo_m[g:g + 1, :]
        best_lo = jnp.max(jnp.where(m2, lo, np.int32(-1)),
                          axis=1, keepdims=True)
        m3 = m2 & (lo == best_lo)
        has_match = best_hi >= np.int32(0)
        seg_hi = jnp.where(onehot, best_hi, seg_hi)
        seg_lo = jnp.where(onehot, best_lo, seg_lo)
        cand = cand & ~(has_match & m3)
    kval = (seg_hi.astype(jnp.float32) * np.float32(4096.0)
            + (seg_lo >> np.int32(5)).astype(jnp.float32)) * np.float32(2.0 ** -36)
    seg = jnp.maximum(kval, np.float32(0.0))
    avg = jnp.sum(jnp.where(lane < np.int32(G - 1), seg, np.float32(0.0)),
                  axis=1, keepdims=True) / np.float32(G - 1)
    out_ref[...] = lax.squeeze(lax.slice(seg, (0, 0), (1, G - 1)), (0,))
    avg_ref[0] = avg[0, 0]


def kernel(pred_labels_mask, gt_labels_mask):
    pred = pred_labels_mask.reshape(-1).astype(jnp.int32)
    gt = gt_labels_mask.reshape(-1).astype(jnp.int32)
    parts = _confusion(gt, pred).reshape(NC, G, P)
    seg_arr, seg_avg1 = pl.pallas_call(
        _greedy_body,
        out_shape=[
            jax.ShapeDtypeStruct((G - 1,), jnp.float32),
            jax.ShapeDtypeStruct((1,), jnp.float32),
        ],
        out_specs=[
            pl.BlockSpec(memory_space=pltpu.MemorySpace.VMEM),
            pl.BlockSpec(memory_space=pltpu.MemorySpace.SMEM),
        ],
    )(parts)
    return (seg_avg1.reshape(()), seg_arr)
